# baseline (speedup 1.0000x reference)
import functools

import jax
import jax.numpy as jnp
from jax.experimental import pallas as pl
from jax.experimental.pallas import tpu as pltpu

_EPS = 1e-5


def _affine_cols(s_ref, g_ref, beta_ref, m):
    mean = s_ref[0:1, :] / m
    var = s_ref[1:2, :] / m - mean * mean
    inv = jax.lax.rsqrt(var + _EPS)
    a = g_ref[...] * inv
    c = beta_ref[...] - mean * a
    return a, c


def _accum_stats(y, s_ref):
    part = jnp.concatenate(
        [jnp.sum(y, axis=0, keepdims=True), jnp.sum(y * y, axis=0, keepdims=True)],
        axis=0,
    )

    @pl.when(pl.program_id(0) == 0)
    def _():
        s_ref[...] = part

    @pl.when(pl.program_id(0) != 0)
    def _():
        s_ref[...] += part


def _layer1_body(x_ref, W_ref, b_ref, y_ref, s_ref):
    xb = x_ref[...].astype(jnp.bfloat16)
    y = jax.nn.relu(
        jnp.dot(xb, W_ref[...], preferred_element_type=jnp.float32) + b_ref[...]
    )
    y_ref[...] = y.astype(jnp.bfloat16)
    _accum_stats(y, s_ref)


def _mid_body(
    y_in_ref, s_in_ref, g_ref, beta_ref, W_ref, b_ref,
    y_ref, s_ref, Wp_ref, bp_ref, *, m,
):
    @pl.when(pl.program_id(0) == 0)
    def _():
        a, c = _affine_cols(s_in_ref, g_ref, beta_ref, m)
        Wp_ref[...] = (W_ref[...] * a.reshape(-1, 1)).astype(jnp.bfloat16)
        bp_ref[...] = (
            jnp.dot(c, W_ref[...], preferred_element_type=jnp.float32) + b_ref[...]
        )

    y = jax.nn.relu(
        jnp.dot(y_in_ref[...], Wp_ref[...], preferred_element_type=jnp.float32)
        + bp_ref[...]
    )
    y_ref[...] = y.astype(jnp.bfloat16)
    _accum_stats(y, s_ref)


def _head_body(
    y1_ref, y2_ref, y3_ref,
    s1_ref, g1_ref, beta1_ref,
    s2_ref, g2_ref, beta2_ref,
    s3_ref, g3_ref, beta3_ref,
    Wl_ref, bl_ref, out_ref, Wp_ref, bp_ref, *, m, h,
):
    @pl.when(pl.program_id(0) == 0)
    def _():
        a1, c1 = _affine_cols(s1_ref, g1_ref, beta1_ref, m)
        a2, c2 = _affine_cols(s2_ref, g2_ref, beta2_ref, m)
        a3, c3 = _affine_cols(s3_ref, g3_ref, beta3_ref, m)
        W1s = Wl_ref[0:h, :]
        W2s = Wl_ref[h : 2 * h, :]
        W3s = Wl_ref[2 * h :, :]
        Wp_ref[0:h, :] = (W1s * a1.reshape(-1, 1)).astype(jnp.bfloat16)
        Wp_ref[h : 2 * h, :] = (W2s * a2.reshape(-1, 1)).astype(jnp.bfloat16)
        Wp_ref[2 * h :, :] = (W3s * a3.reshape(-1, 1)).astype(jnp.bfloat16)
        bp_ref[...] = (
            jnp.dot(c1, W1s, preferred_element_type=jnp.float32)
            + jnp.dot(c2, W2s, preferred_element_type=jnp.float32)
            + jnp.dot(c3, W3s, preferred_element_type=jnp.float32)
            + bl_ref[...]
        )

    acc = jnp.dot(y1_ref[...], Wp_ref[0:h, :], preferred_element_type=jnp.float32)
    acc += jnp.dot(y2_ref[...], Wp_ref[h : 2 * h, :], preferred_element_type=jnp.float32)
    acc += jnp.dot(y3_ref[...], Wp_ref[2 * h :, :], preferred_element_type=jnp.float32)
    out_ref[...] = jax.nn.relu(acc + bp_ref[...])


def _full(shape):
    return pl.BlockSpec(shape, lambda i: (0, 0))


def _rows(r, c):
    return pl.BlockSpec((r, c), lambda i: (i, 0))


def kernel(x, adj, W1, b1, g1, beta1, W2, b2, g2, beta2, W3, b3, g3, beta3, Wl, bl):
    del adj
    B, N, Cin = x.shape
    H = W1.shape[1]
    Cout = W3.shape[1]
    M = B * N
    R = 2048
    nb = M // R
    grid = (nb,)
    mf = float(M)

    xf = x.reshape(M, Cin)
    row = lambda v: v.reshape(1, -1)

    y1, s1 = pl.pallas_call(
        _layer1_body,
        grid=grid,
        in_specs=[_rows(R, Cin), _full((Cin, H)), _full((1, H))],
        out_specs=[_rows(R, H), _full((2, H))],
        out_shape=[
            jax.ShapeDtypeStruct((M, H), jnp.bfloat16),
            jax.ShapeDtypeStruct((2, H), jnp.float32),
        ],
    )(xf, W1.astype(jnp.bfloat16), row(b1))

    mid = functools.partial(_mid_body, m=mf)
    y2, s2 = pl.pallas_call(
        mid,
        grid=grid,
        in_specs=[_rows(R, H), _full((2, H)), _full((1, H)), _full((1, H)),
                  _full((H, H)), _full((1, H))],
        out_specs=[_rows(R, H), _full((2, H))],
        out_shape=[
            jax.ShapeDtypeStruct((M, H), jnp.bfloat16),
            jax.ShapeDtypeStruct((2, H), jnp.float32),
        ],
        scratch_shapes=[
            pltpu.VMEM((H, H), jnp.bfloat16),
            pltpu.VMEM((1, H), jnp.float32),
        ],
    )(y1, s1, row(g1), row(beta1), W2, row(b2))

    y3, s3 = pl.pallas_call(
        mid,
        grid=grid,
        in_specs=[_rows(R, H), _full((2, H)), _full((1, H)), _full((1, H)),
                  _full((H, Cout)), _full((1, Cout))],
        out_specs=[_rows(R, Cout), _full((2, Cout))],
        out_shape=[
            jax.ShapeDtypeStruct((M, Cout), jnp.bfloat16),
            jax.ShapeDtypeStruct((2, Cout), jnp.float32),
        ],
        scratch_shapes=[
            pltpu.VMEM((H, Cout), jnp.bfloat16),
            pltpu.VMEM((1, Cout), jnp.float32),
        ],
    )(y2, s2, row(g2), row(beta2), W3, row(b3))

    out = pl.pallas_call(
        functools.partial(_head_body, m=mf, h=H),
        grid=grid,
        in_specs=[
            _rows(R, H), _rows(R, H), _rows(R, Cout),
            _full((2, H)), _full((1, H)), _full((1, H)),
            _full((2, H)), _full((1, H)), _full((1, H)),
            _full((2, Cout)), _full((1, Cout)), _full((1, Cout)),
            _full((2 * H + Cout, Cout)), _full((1, Cout)),
        ],
        out_specs=_rows(R, Cout),
        out_shape=jax.ShapeDtypeStruct((M, Cout), jnp.float32),
        scratch_shapes=[
            pltpu.VMEM((2 * H + Cout, Cout), jnp.bfloat16),
            pltpu.VMEM((1, Cout), jnp.float32),
        ],
    )(
        y1, y2, y3,
        s1, row(g1), row(beta1),
        s2, row(g2), row(beta2),
        s3, row(g3), row(beta3),
        Wl, row(bl),
    )

    return out.reshape(B, N, Cout)


# speedup vs baseline: 1.4986x; 1.4986x over previous
import functools

import jax
import jax.numpy as jnp
from jax.experimental import pallas as pl
from jax.experimental.pallas import tpu as pltpu

_EPS = 1e-5


def _affine_cols(s_scr, g_ref, beta_ref, m):
    mean = s_scr[0:1, :] / m
    var = s_scr[1:2, :] / m - mean * mean
    inv = jax.lax.rsqrt(var + _EPS)
    a = g_ref[...] * inv
    c = beta_ref[...] - mean * a
    return a, c


def _accum_stats(y, s_scr, i):
    part = jnp.concatenate(
        [jnp.sum(y, axis=0, keepdims=True), jnp.sum(y * y, axis=0, keepdims=True)],
        axis=0,
    )

    @pl.when(i == 0)
    def _():
        s_scr[...] = part

    @pl.when(i != 0)
    def _():
        s_scr[...] += part


def _fold(W, a, c, b):
    Wp = (W * a.reshape(-1, 1)).astype(jnp.bfloat16)
    bp = jnp.dot(c, W, preferred_element_type=jnp.float32) + b
    return Wp, bp


def _fused_body(
    x_ref, W1_ref, b1_ref, g1_ref, beta1_ref, W2_ref, b2_ref, g2_ref, beta2_ref,
    W3_ref, b3_ref, g3_ref, beta3_ref, Wl_ref, bl_ref,
    out_ref,
    y1_scr, y2_scr, y3_scr, s1_scr, s2_scr, s3_scr,
    Wp2_scr, bp2_scr, Wp3_scr, bp3_scr, Wpl_scr, bpl_scr,
    *, m, h, r,
):
    s = pl.program_id(0)
    i = pl.program_id(1)
    rows = pl.ds(i * r, r)

    @pl.when(s == 0)
    def _():
        xb = x_ref[...].astype(jnp.bfloat16)
        y = jax.nn.relu(
            jnp.dot(xb, W1_ref[...], preferred_element_type=jnp.float32)
            + b1_ref[...]
        )
        y1_scr[rows, :] = y.astype(jnp.bfloat16)
        _accum_stats(y, s1_scr, i)

    @pl.when((s == 1) & (i == 0))
    def _():
        a, c = _affine_cols(s1_scr, g1_ref, beta1_ref, m)
        Wp2_scr[...], bp2_scr[...] = _fold(W2_ref[...], a, c, b2_ref[...])

    @pl.when(s == 1)
    def _():
        y = jax.nn.relu(
            jnp.dot(y1_scr[rows, :], Wp2_scr[...], preferred_element_type=jnp.float32)
            + bp2_scr[...]
        )
        y2_scr[rows, :] = y.astype(jnp.bfloat16)
        _accum_stats(y, s2_scr, i)

    @pl.when((s == 2) & (i == 0))
    def _():
        a, c = _affine_cols(s2_scr, g2_ref, beta2_ref, m)
        Wp3_scr[...], bp3_scr[...] = _fold(W3_ref[...], a, c, b3_ref[...])

    @pl.when(s == 2)
    def _():
        y = jax.nn.relu(
            jnp.dot(y2_scr[rows, :], Wp3_scr[...], preferred_element_type=jnp.float32)
            + bp3_scr[...]
        )
        y3_scr[rows, :] = y.astype(jnp.bfloat16)
        _accum_stats(y, s3_scr, i)

    @pl.when((s == 3) & (i == 0))
    def _():
        a1, c1 = _affine_cols(s1_scr, g1_ref, beta1_ref, m)
        a2, c2 = _affine_cols(s2_scr, g2_ref, beta2_ref, m)
        a3, c3 = _affine_cols(s3_scr, g3_ref, beta3_ref, m)
        Wp1, bp1 = _fold(Wl_ref[0:h, :], a1, c1, bl_ref[...])
        Wp2, bp2 = _fold(Wl_ref[h : 2 * h, :], a2, c2, bp1)
        Wp3, bp3 = _fold(Wl_ref[2 * h :, :], a3, c3, bp2)
        Wpl_scr[0:h, :] = Wp1
        Wpl_scr[h : 2 * h, :] = Wp2
        Wpl_scr[2 * h :, :] = Wp3
        bpl_scr[...] = bp3

    @pl.when(s == 3)
    def _():
        acc = jnp.dot(
            y1_scr[rows, :], Wpl_scr[0:h, :], preferred_element_type=jnp.float32
        )
        acc += jnp.dot(
            y2_scr[rows, :], Wpl_scr[h : 2 * h, :], preferred_element_type=jnp.float32
        )
        acc += jnp.dot(
            y3_scr[rows, :], Wpl_scr[2 * h :, :], preferred_element_type=jnp.float32
        )
        out_ref[...] = jax.nn.relu(acc + bpl_scr[...])


def kernel(x, adj, W1, b1, g1, beta1, W2, b2, g2, beta2, W3, b3, g3, beta3, Wl, bl):
    del adj
    B, N, Cin = x.shape
    H = W1.shape[1]
    Cout = W3.shape[1]
    M = B * N
    R = 2048
    nb = M // R
    mf = float(M)

    xf = x.reshape(M, Cin)
    row = lambda v: v.reshape(1, -1)
    full = lambda shape: pl.BlockSpec(shape, lambda s, i: (0, 0))

    out = pl.pallas_call(
        functools.partial(_fused_body, m=mf, h=H, r=R),
        grid=(4, nb),
        in_specs=[
            pl.BlockSpec((R, Cin), lambda s, i: (jnp.where(s == 0, i, 0), 0)),
            full((Cin, H)), full((1, H)), full((1, H)), full((1, H)),
            full((H, H)), full((1, H)), full((1, H)), full((1, H)),
            full((H, Cout)), full((1, Cout)), full((1, Cout)), full((1, Cout)),
            full((2 * H + Cout, Cout)), full((1, Cout)),
        ],
        out_specs=pl.BlockSpec((R, Cout), lambda s, i: (jnp.where(s == 3, i, 0), 0)),
        out_shape=jax.ShapeDtypeStruct((M, Cout), jnp.float32),
        scratch_shapes=[
            pltpu.VMEM((M, H), jnp.bfloat16),
            pltpu.VMEM((M, H), jnp.bfloat16),
            pltpu.VMEM((M, Cout), jnp.bfloat16),
            pltpu.VMEM((2, H), jnp.float32),
            pltpu.VMEM((2, H), jnp.float32),
            pltpu.VMEM((2, Cout), jnp.float32),
            pltpu.VMEM((H, H), jnp.bfloat16),
            pltpu.VMEM((1, H), jnp.float32),
            pltpu.VMEM((H, Cout), jnp.bfloat16),
            pltpu.VMEM((1, Cout), jnp.float32),
            pltpu.VMEM((2 * H + Cout, Cout), jnp.bfloat16),
            pltpu.VMEM((1, Cout), jnp.float32),
        ],
    )(
        xf, W1.astype(jnp.bfloat16), row(b1), row(g1), row(beta1),
        W2, row(b2), row(g2), row(beta2),
        W3, row(b3), row(g3), row(beta3),
        Wl, row(bl),
    )

    return out.reshape(B, N, Cout)


# speedup vs baseline: 1.5515x; 1.0353x over previous
import functools

import jax
import jax.numpy as jnp
from jax.experimental import pallas as pl
from jax.experimental.pallas import tpu as pltpu

_EPS = 1e-5


def _affine_cols(s_scr, g_ref, beta_ref, m):
    mean = s_scr[0:1, :] / m
    var = s_scr[1:2, :] / m - mean * mean
    inv = jax.lax.rsqrt(var + _EPS)
    a = g_ref[...] * inv
    c = beta_ref[...] - mean * a
    return a, c


def _accum_stats(y, s_scr, i):
    part = jnp.concatenate(
        [jnp.sum(y, axis=0, keepdims=True), jnp.sum(y * y, axis=0, keepdims=True)],
        axis=0,
    )

    @pl.when(i == 0)
    def _():
        s_scr[...] = part

    @pl.when(i != 0)
    def _():
        s_scr[...] += part


def _fold(W, a, c, b):
    Wp = (W * a.reshape(-1, 1)).astype(jnp.bfloat16)
    bp = jnp.dot(c, W, preferred_element_type=jnp.float32) + b
    return Wp, bp


def _fused_body(
    x_ref, W1_ref, b1_ref, g1_ref, beta1_ref, W2_ref, b2_ref, g2_ref, beta2_ref,
    W3_ref, b3_ref, g3_ref, beta3_ref, Wl_ref, bl_ref,
    out_ref,
    y1_scr, y2_scr, y3_scr, s1_scr, s2_scr, s3_scr,
    Wp2_scr, bp2_scr, Wp3_scr, bp3_scr, Wpl_scr, bpl_scr,
    *, m, h, r,
):
    s = pl.program_id(0)
    i = pl.program_id(1)
    rows = pl.ds(i * r, r)

    @pl.when(s == 0)
    def _():
        y = jax.nn.relu(
            jnp.dot(x_ref[...], W1_ref[...], preferred_element_type=jnp.float32)
            + b1_ref[...]
        )
        y1_scr[rows, :] = y.astype(jnp.bfloat16)
        _accum_stats(y, s1_scr, i)

    @pl.when((s == 1) & (i == 0))
    def _():
        a, c = _affine_cols(s1_scr, g1_ref, beta1_ref, m)
        Wp2_scr[...], bp2_scr[...] = _fold(W2_ref[...], a, c, b2_ref[...])

    @pl.when(s == 1)
    def _():
        y = jax.nn.relu(
            jnp.dot(y1_scr[rows, :], Wp2_scr[...], preferred_element_type=jnp.float32)
            + bp2_scr[...]
        )
        y2_scr[rows, :] = y.astype(jnp.bfloat16)
        _accum_stats(y, s2_scr, i)

    @pl.when((s == 2) & (i == 0))
    def _():
        a, c = _affine_cols(s2_scr, g2_ref, beta2_ref, m)
        Wp3_scr[...], bp3_scr[...] = _fold(W3_ref[...], a, c, b3_ref[...])

    @pl.when(s == 2)
    def _():
        y = jax.nn.relu(
            jnp.dot(y2_scr[rows, :], Wp3_scr[...], preferred_element_type=jnp.float32)
            + bp3_scr[...]
        )
        y3_scr[rows, :] = y.astype(jnp.bfloat16)
        _accum_stats(y, s3_scr, i)

    @pl.when((s == 3) & (i == 0))
    def _():
        a1, c1 = _affine_cols(s1_scr, g1_ref, beta1_ref, m)
        a2, c2 = _affine_cols(s2_scr, g2_ref, beta2_ref, m)
        a3, c3 = _affine_cols(s3_scr, g3_ref, beta3_ref, m)
        Wp1, bp1 = _fold(Wl_ref[0:h, :], a1, c1, bl_ref[...])
        Wp2, bp2 = _fold(Wl_ref[h : 2 * h, :], a2, c2, bp1)
        Wp3, bp3 = _fold(Wl_ref[2 * h :, :], a3, c3, bp2)
        Wpl_scr[0:h, :] = Wp1
        Wpl_scr[h : 2 * h, :] = Wp2
        Wpl_scr[2 * h :, :] = Wp3
        bpl_scr[...] = bp3

    @pl.when(s == 3)
    def _():
        acc = jnp.dot(
            y1_scr[rows, :], Wpl_scr[0:h, :], preferred_element_type=jnp.float32
        )
        acc += jnp.dot(
            y2_scr[rows, :], Wpl_scr[h : 2 * h, :], preferred_element_type=jnp.float32
        )
        acc += jnp.dot(
            y3_scr[rows, :], Wpl_scr[2 * h :, :], preferred_element_type=jnp.float32
        )
        out_ref[...] = jax.nn.relu(acc + bpl_scr[...])


def kernel(x, adj, W1, b1, g1, beta1, W2, b2, g2, beta2, W3, b3, g3, beta3, Wl, bl):
    del adj
    B, N, Cin = x.shape
    H = W1.shape[1]
    Cout = W3.shape[1]
    M = B * N
    R = 4096
    nb = M // R
    mf = float(M)

    xf = x.reshape(M, Cin).astype(jnp.bfloat16)
    row = lambda v: v.reshape(1, -1)
    full = lambda shape: pl.BlockSpec(shape, lambda s, i: (0, 0))

    out = pl.pallas_call(
        functools.partial(_fused_body, m=mf, h=H, r=R),
        grid=(4, nb),
        in_specs=[
            pl.BlockSpec((R, Cin), lambda s, i: (jnp.where(s == 0, i, 0), 0)),
            full((Cin, H)), full((1, H)), full((1, H)), full((1, H)),
            full((H, H)), full((1, H)), full((1, H)), full((1, H)),
            full((H, Cout)), full((1, Cout)), full((1, Cout)), full((1, Cout)),
            full((2 * H + Cout, Cout)), full((1, Cout)),
        ],
        out_specs=pl.BlockSpec((R, Cout), lambda s, i: (jnp.where(s == 3, i, 0), 0)),
        out_shape=jax.ShapeDtypeStruct((M, Cout), jnp.float32),
        scratch_shapes=[
            pltpu.VMEM((M, H), jnp.bfloat16),
            pltpu.VMEM((M, H), jnp.bfloat16),
            pltpu.VMEM((M, Cout), jnp.bfloat16),
            pltpu.VMEM((2, H), jnp.float32),
            pltpu.VMEM((2, H), jnp.float32),
            pltpu.VMEM((2, Cout), jnp.float32),
            pltpu.VMEM((H, H), jnp.bfloat16),
            pltpu.VMEM((1, H), jnp.float32),
            pltpu.VMEM((H, Cout), jnp.bfloat16),
            pltpu.VMEM((1, Cout), jnp.float32),
            pltpu.VMEM((2 * H + Cout, Cout), jnp.bfloat16),
            pltpu.VMEM((1, Cout), jnp.float32),
        ],
    )(
        xf, W1.astype(jnp.bfloat16), row(b1), row(g1), row(beta1),
        W2, row(b2), row(g2), row(beta2),
        W3, row(b3), row(g3), row(beta3),
        Wl, row(bl),
    )

    return out.reshape(B, N, Cout)


# speedup vs baseline: 1.8455x; 1.1895x over previous
import functools

import jax
import jax.numpy as jnp
from jax.experimental import pallas as pl
from jax.experimental.pallas import tpu as pltpu

_EPS = 1e-5


def _affine_cols(s_scr, g_ref, beta_ref, m):
    mean = s_scr[0:1, :] / m
    var = s_scr[1:2, :] / m - mean * mean
    inv = jax.lax.rsqrt(var + _EPS)
    a = g_ref[...] * inv
    c = beta_ref[...] - mean * a
    return a, c


def _accum_stats(y, s_scr, i):
    part = jnp.concatenate(
        [jnp.sum(y, axis=0, keepdims=True), jnp.sum(y * y, axis=0, keepdims=True)],
        axis=0,
    )

    @pl.when(i == 0)
    def _():
        s_scr[...] = part

    @pl.when(i != 0)
    def _():
        s_scr[...] += part


def _fold(W, a, c, b):
    Wp = (W * a.reshape(-1, 1)).astype(jnp.bfloat16)
    bp = jnp.dot(c, W, preferred_element_type=jnp.float32) + b
    return Wp, bp


def _fused_body(
    x_ref, W1_ref, b1_ref, g1_ref, beta1_ref, W2_ref, b2_ref, g2_ref, beta2_ref,
    W3_ref, b3_ref, g3_ref, beta3_ref, Wl_ref, bl_ref,
    out_ref,
    y1_scr, y2_scr, y3_scr, s1_scr, s2_scr, s3_scr,
    Wp2_scr, bp2_scr, Wp3_scr, bp3_scr, Wpl_scr, bpl_scr,
    *, m, h, r,
):
    s = pl.program_id(0)
    i = pl.program_id(1)
    rows = pl.ds(i * r, r)

    @pl.when(s == 0)
    def _():
        xb = x_ref[...].astype(jnp.bfloat16)
        y = jax.nn.relu(
            jnp.dot(xb, W1_ref[...], preferred_element_type=jnp.float32)
            + b1_ref[...]
        )
        y1_scr[rows, :] = y.astype(jnp.bfloat16)
        _accum_stats(y, s1_scr, i)

    @pl.when((s == 1) & (i == 0))
    def _():
        a, c = _affine_cols(s1_scr, g1_ref, beta1_ref, m)
        Wp2_scr[...], bp2_scr[...] = _fold(W2_ref[...], a, c, b2_ref[...])

    @pl.when(s == 1)
    def _():
        y = jax.nn.relu(
            jnp.dot(y1_scr[rows, :], Wp2_scr[...], preferred_element_type=jnp.float32)
            + bp2_scr[...]
        )
        y2_scr[rows, :] = y.astype(jnp.bfloat16)
        _accum_stats(y, s2_scr, i)

    @pl.when((s == 2) & (i == 0))
    def _():
        a, c = _affine_cols(s2_scr, g2_ref, beta2_ref, m)
        Wp3_scr[...], bp3_scr[...] = _fold(W3_ref[...], a, c, b3_ref[...])

    @pl.when(s == 2)
    def _():
        y = jax.nn.relu(
            jnp.dot(y2_scr[rows, :], Wp3_scr[...], preferred_element_type=jnp.float32)
            + bp3_scr[...]
        )
        y3_scr[rows, :] = y.astype(jnp.bfloat16)
        _accum_stats(y, s3_scr, i)

    @pl.when((s == 3) & (i == 0))
    def _():
        a1, c1 = _affine_cols(s1_scr, g1_ref, beta1_ref, m)
        a2, c2 = _affine_cols(s2_scr, g2_ref, beta2_ref, m)
        a3, c3 = _affine_cols(s3_scr, g3_ref, beta3_ref, m)
        Wp1, bp1 = _fold(Wl_ref[0:h, :], a1, c1, bl_ref[...])
        Wp2, bp2 = _fold(Wl_ref[h : 2 * h, :], a2, c2, bp1)
        Wp3, bp3 = _fold(Wl_ref[2 * h :, :], a3, c3, bp2)
        Wpl_scr[0:h, :] = Wp1
        Wpl_scr[h : 2 * h, :] = Wp2
        Wpl_scr[2 * h :, :] = Wp3
        bpl_scr[...] = bp3

    @pl.when(s == 3)
    def _():
        acc = jnp.dot(
            y1_scr[rows, :], Wpl_scr[0:h, :], preferred_element_type=jnp.float32
        )
        acc += jnp.dot(
            y2_scr[rows, :], Wpl_scr[h : 2 * h, :], preferred_element_type=jnp.float32
        )
        acc += jnp.dot(
            y3_scr[rows, :], Wpl_scr[2 * h :, :], preferred_element_type=jnp.float32
        )
        out_ref[...] = jax.nn.relu(acc + bpl_scr[...])


def kernel(x, adj, W1, b1, g1, beta1, W2, b2, g2, beta2, W3, b3, g3, beta3, Wl, bl):
    del adj
    B, N, Cin = x.shape
    H = W1.shape[1]
    Cout = W3.shape[1]
    M = B * N
    R = 4096
    nb = M // R
    mf = float(M)

    xf = x.reshape(M, Cin)
    row = lambda v: v.reshape(1, -1)
    full = lambda shape: pl.BlockSpec(shape, lambda s, i: (0, 0))

    out = pl.pallas_call(
        functools.partial(_fused_body, m=mf, h=H, r=R),
        grid=(4, nb),
        in_specs=[
            pl.BlockSpec((R, Cin), lambda s, i: (jnp.where(s == 0, i, 0), 0)),
            full((Cin, H)), full((1, H)), full((1, H)), full((1, H)),
            full((H, H)), full((1, H)), full((1, H)), full((1, H)),
            full((H, Cout)), full((1, Cout)), full((1, Cout)), full((1, Cout)),
            full((2 * H + Cout, Cout)), full((1, Cout)),
        ],
        out_specs=pl.BlockSpec((R, Cout), lambda s, i: (jnp.where(s == 3, i, 0), 0)),
        out_shape=jax.ShapeDtypeStruct((M, Cout), jnp.float32),
        scratch_shapes=[
            pltpu.VMEM((M, H), jnp.bfloat16),
            pltpu.VMEM((M, H), jnp.bfloat16),
            pltpu.VMEM((M, Cout), jnp.bfloat16),
            pltpu.VMEM((2, H), jnp.float32),
            pltpu.VMEM((2, H), jnp.float32),
            pltpu.VMEM((2, Cout), jnp.float32),
            pltpu.VMEM((H, H), jnp.bfloat16),
            pltpu.VMEM((1, H), jnp.float32),
            pltpu.VMEM((H, Cout), jnp.bfloat16),
            pltpu.VMEM((1, Cout), jnp.float32),
            pltpu.VMEM((2 * H + Cout, Cout), jnp.bfloat16),
            pltpu.VMEM((1, Cout), jnp.float32),
        ],
    )(
        xf, W1.astype(jnp.bfloat16), row(b1), row(g1), row(beta1),
        W2, row(b2), row(g2), row(beta2),
        W3, row(b3), row(g3), row(beta3),
        Wl, row(bl),
    )

    return out.reshape(B, N, Cout)


# speedup vs baseline: 1.9829x; 1.0745x over previous
import functools

import jax
import jax.numpy as jnp
from jax.experimental import pallas as pl
from jax.experimental.pallas import tpu as pltpu

_EPS = 1e-5


def _affine_cols(s_scr, g_ref, beta_ref, m):
    mean = s_scr[0:1, :] / m
    var = s_scr[1:2, :] / m - mean * mean
    inv = jax.lax.rsqrt(var + _EPS)
    a = g_ref[...] * inv
    c = beta_ref[...] - mean * a
    return a, c


def _accum_stats(y, s_scr, i):
    part = jnp.concatenate(
        [jnp.sum(y, axis=0, keepdims=True), jnp.sum(y * y, axis=0, keepdims=True)],
        axis=0,
    )

    @pl.when(i == 0)
    def _():
        s_scr[...] = part

    @pl.when(i != 0)
    def _():
        s_scr[...] += part


def _fold(W, a, c, b):
    Wp = (W * a.reshape(-1, 1)).astype(jnp.bfloat16)
    bp = jnp.dot(c, W, preferred_element_type=jnp.float32) + b
    return Wp, bp


def _fused_body(
    x_ref, W1_ref, b1_ref, g1_ref, beta1_ref, W2_ref, b2_ref, g2_ref, beta2_ref,
    W3_ref, b3_ref, g3_ref, beta3_ref, Wl_ref, bl_ref,
    out_ref,
    y1_scr, y2_scr, y3_scr, s1_scr, s2_scr, s3_scr,
    Wp2_scr, bp2_scr, Wp3_scr, bp3_scr, Wpl_scr, bpl_scr,
    *, m, h, r,
):
    s = pl.program_id(0)
    i = pl.program_id(1)
    rows = pl.ds(i * r, r)

    @pl.when(s == 0)
    def _():
        xb = x_ref[...].astype(jnp.bfloat16)
        y = jax.nn.relu(
            jnp.dot(xb, W1_ref[...], preferred_element_type=jnp.float32)
            + b1_ref[...]
        )
        y1_scr[rows, :] = y.astype(jnp.bfloat16)
        _accum_stats(y, s1_scr, i)

    @pl.when((s == 1) & (i == 0))
    def _():
        a, c = _affine_cols(s1_scr, g1_ref, beta1_ref, m)
        Wp2_scr[...], bp2_scr[...] = _fold(W2_ref[...], a, c, b2_ref[...])

    @pl.when(s == 1)
    def _():
        y = jax.nn.relu(
            jnp.dot(y1_scr[rows, :], Wp2_scr[...], preferred_element_type=jnp.float32)
            + bp2_scr[...]
        )
        y2_scr[rows, :] = y.astype(jnp.bfloat16)
        _accum_stats(y, s2_scr, i)

    @pl.when((s == 2) & (i == 0))
    def _():
        a, c = _affine_cols(s2_scr, g2_ref, beta2_ref, m)
        Wp3_scr[...], bp3_scr[...] = _fold(W3_ref[...], a, c, b3_ref[...])

    @pl.when(s == 2)
    def _():
        y = jax.nn.relu(
            jnp.dot(y2_scr[rows, :], Wp3_scr[...], preferred_element_type=jnp.float32)
            + bp3_scr[...]
        )
        y3_scr[rows, :] = y.astype(jnp.bfloat16)
        _accum_stats(y, s3_scr, i)

    @pl.when((s == 3) & (i == 0))
    def _():
        a1, c1 = _affine_cols(s1_scr, g1_ref, beta1_ref, m)
        a2, c2 = _affine_cols(s2_scr, g2_ref, beta2_ref, m)
        a3, c3 = _affine_cols(s3_scr, g3_ref, beta3_ref, m)
        Wp1, bp1 = _fold(Wl_ref[0:h, :], a1, c1, bl_ref[...])
        Wp2, bp2 = _fold(Wl_ref[h : 2 * h, :], a2, c2, bp1)
        Wp3, bp3 = _fold(Wl_ref[2 * h :, :], a3, c3, bp2)
        Wpl_scr[0:h, :] = Wp1
        Wpl_scr[h : 2 * h, :] = Wp2
        Wpl_scr[2 * h :, :] = Wp3
        bpl_scr[...] = bp3

    @pl.when(s == 3)
    def _():
        acc = jnp.dot(
            y1_scr[rows, :], Wpl_scr[0:h, :], preferred_element_type=jnp.float32
        )
        acc += jnp.dot(
            y2_scr[rows, :], Wpl_scr[h : 2 * h, :], preferred_element_type=jnp.float32
        )
        acc += jnp.dot(
            y3_scr[rows, :], Wpl_scr[2 * h :, :], preferred_element_type=jnp.float32
        )
        out_ref[...] = jax.nn.relu(acc + bpl_scr[...])


def kernel(x, adj, W1, b1, g1, beta1, W2, b2, g2, beta2, W3, b3, g3, beta3, Wl, bl):
    del adj
    B, N, Cin = x.shape
    H = W1.shape[1]
    Cout = W3.shape[1]
    M = B * N
    R = 10240
    nb = M // R
    mf = float(M)

    xf = x.reshape(M, Cin)
    row = lambda v: v.reshape(1, -1)
    full = lambda shape: pl.BlockSpec(shape, lambda s, i: (0, 0))

    out = pl.pallas_call(
        functools.partial(_fused_body, m=mf, h=H, r=R),
        grid=(4, nb),
        in_specs=[
            pl.BlockSpec((R, Cin), lambda s, i: (jnp.where(s == 0, i, 0), 0)),
            full((Cin, H)), full((1, H)), full((1, H)), full((1, H)),
            full((H, H)), full((1, H)), full((1, H)), full((1, H)),
            full((H, Cout)), full((1, Cout)), full((1, Cout)), full((1, Cout)),
            full((2 * H + Cout, Cout)), full((1, Cout)),
        ],
        out_specs=pl.BlockSpec((R, Cout), lambda s, i: (jnp.where(s == 3, i, 0), 0)),
        out_shape=jax.ShapeDtypeStruct((M, Cout), jnp.float32),
        scratch_shapes=[
            pltpu.VMEM((M, H), jnp.bfloat16),
            pltpu.VMEM((M, H), jnp.bfloat16),
            pltpu.VMEM((M, Cout), jnp.bfloat16),
            pltpu.VMEM((2, H), jnp.float32),
            pltpu.VMEM((2, H), jnp.float32),
            pltpu.VMEM((2, Cout), jnp.float32),
            pltpu.VMEM((H, H), jnp.bfloat16),
            pltpu.VMEM((1, H), jnp.float32),
            pltpu.VMEM((H, Cout), jnp.bfloat16),
            pltpu.VMEM((1, Cout), jnp.float32),
            pltpu.VMEM((2 * H + Cout, Cout), jnp.bfloat16),
            pltpu.VMEM((1, Cout), jnp.float32),
        ],
    )(
        xf, W1.astype(jnp.bfloat16), row(b1), row(g1), row(beta1),
        W2, row(b2), row(g2), row(beta2),
        W3, row(b3), row(g3), row(beta3),
        Wl, row(bl),
    )

    return out.reshape(B, N, Cout)


# speedup vs baseline: 2.1498x; 1.0841x over previous
import functools

import jax
import jax.numpy as jnp
from jax.experimental import pallas as pl
from jax.experimental.pallas import tpu as pltpu

_EPS = 1e-5


def _affine_cols(s_scr, g_ref, beta_ref, m):
    mean = s_scr[0:1, :] / m
    var = s_scr[1:2, :] / m - mean * mean
    inv = jax.lax.rsqrt(var + _EPS)
    a = g_ref[...] * inv
    c = beta_ref[...] - mean * a
    return a, c


def _accum_stats(y, s_scr, i):
    part = jnp.concatenate(
        [jnp.sum(y, axis=0, keepdims=True), jnp.sum(y * y, axis=0, keepdims=True)],
        axis=0,
    )

    @pl.when(i == 0)
    def _():
        s_scr[...] = part

    @pl.when(i != 0)
    def _():
        s_scr[...] += part


def _fused_body(
    x_ref, W1_ref, b1_ref, g1_ref, beta1_ref, W2_ref, b2_ref, g2_ref, beta2_ref,
    W3_ref, b3_ref, g3_ref, beta3_ref, Wl_ref, bl_ref,
    out_ref,
    y1_scr, y2_scr, y3_scr, acc_scr, s1_scr, s2_scr, s3_scr,
    Wc1_scr, bp2_scr, Wc2_scr, bp3_scr, Wp3l_scr, bpl_scr,
    *, m, h, co, r,
):
    s = pl.program_id(0)
    i = pl.program_id(1)
    rows = pl.ds(i * r, r)

    @pl.when(s == 0)
    def _():
        xb = x_ref[...].astype(jnp.bfloat16)
        y = jax.nn.relu(
            jnp.dot(xb, W1_ref[...], preferred_element_type=jnp.float32)
            + b1_ref[...]
        )
        y1_scr[rows, :] = y.astype(jnp.bfloat16)
        _accum_stats(y, s1_scr, i)

    @pl.when((s == 1) & (i == 0))
    def _():
        a, c = _affine_cols(s1_scr, g1_ref, beta1_ref, m)
        av = a.reshape(-1, 1)
        Wc1_scr[:, 0:h] = (W2_ref[...] * av).astype(jnp.bfloat16)
        Wc1_scr[:, h:] = (Wl_ref[0:h, :] * av).astype(jnp.bfloat16)
        bp2_scr[...] = (
            jnp.dot(c, W2_ref[...], preferred_element_type=jnp.float32)
            + b2_ref[...]
        )

    @pl.when(s == 1)
    def _():
        z = jnp.dot(
            y1_scr[rows, :], Wc1_scr[...], preferred_element_type=jnp.float32
        )
        y = jax.nn.relu(z[:, 0:h] + bp2_scr[...])
        y2_scr[rows, :] = y.astype(jnp.bfloat16)
        acc_scr[rows, :] = z[:, h:]
        _accum_stats(y, s2_scr, i)

    @pl.when((s == 2) & (i == 0))
    def _():
        a, c = _affine_cols(s2_scr, g2_ref, beta2_ref, m)
        av = a.reshape(-1, 1)
        Wc2_scr[:, 0:co] = (W3_ref[...] * av).astype(jnp.bfloat16)
        Wc2_scr[:, co:] = (Wl_ref[h : 2 * h, :] * av).astype(jnp.bfloat16)
        bp3_scr[...] = (
            jnp.dot(c, W3_ref[...], preferred_element_type=jnp.float32)
            + b3_ref[...]
        )

    @pl.when(s == 2)
    def _():
        z = jnp.dot(
            y2_scr[rows, :], Wc2_scr[...], preferred_element_type=jnp.float32
        )
        y = jax.nn.relu(z[:, 0:co] + bp3_scr[...])
        y3_scr[rows, :] = y.astype(jnp.bfloat16)
        acc_scr[rows, :] += z[:, co:]
        _accum_stats(y, s3_scr, i)

    @pl.when((s == 3) & (i == 0))
    def _():
        a1, c1 = _affine_cols(s1_scr, g1_ref, beta1_ref, m)
        a2, c2 = _affine_cols(s2_scr, g2_ref, beta2_ref, m)
        a3, c3 = _affine_cols(s3_scr, g3_ref, beta3_ref, m)
        Wp3l_scr[...] = (Wl_ref[2 * h :, :] * a3.reshape(-1, 1)).astype(jnp.bfloat16)
        bpl_scr[...] = (
            jnp.dot(c1, Wl_ref[0:h, :], preferred_element_type=jnp.float32)
            + jnp.dot(c2, Wl_ref[h : 2 * h, :], preferred_element_type=jnp.float32)
            + jnp.dot(c3, Wl_ref[2 * h :, :], preferred_element_type=jnp.float32)
            + bl_ref[...]
        )

    @pl.when(s == 3)
    def _():
        z = jnp.dot(
            y3_scr[rows, :], Wp3l_scr[...], preferred_element_type=jnp.float32
        )
        out_ref[...] = jax.nn.relu(acc_scr[rows, :] + z + bpl_scr[...])


def kernel(x, adj, W1, b1, g1, beta1, W2, b2, g2, beta2, W3, b3, g3, beta3, Wl, bl):
    del adj
    B, N, Cin = x.shape
    H = W1.shape[1]
    Cout = W3.shape[1]
    M = B * N
    R = 5120
    nb = M // R
    mf = float(M)

    xf = x.reshape(M, Cin)
    row = lambda v: v.reshape(1, -1)
    full = lambda shape: pl.BlockSpec(shape, lambda s, i: (0, 0))

    out = pl.pallas_call(
        functools.partial(_fused_body, m=mf, h=H, co=Cout, r=R),
        grid=(4, nb),
        in_specs=[
            pl.BlockSpec((R, Cin), lambda s, i: (jnp.where(s == 0, i, 0), 0)),
            full((Cin, H)), full((1, H)), full((1, H)), full((1, H)),
            full((H, H)), full((1, H)), full((1, H)), full((1, H)),
            full((H, Cout)), full((1, Cout)), full((1, Cout)), full((1, Cout)),
            full((2 * H + Cout, Cout)), full((1, Cout)),
        ],
        out_specs=pl.BlockSpec((R, Cout), lambda s, i: (jnp.where(s == 3, i, 0), 0)),
        out_shape=jax.ShapeDtypeStruct((M, Cout), jnp.float32),
        scratch_shapes=[
            pltpu.VMEM((M, H), jnp.bfloat16),
            pltpu.VMEM((M, H), jnp.bfloat16),
            pltpu.VMEM((M, Cout), jnp.bfloat16),
            pltpu.VMEM((M, Cout), jnp.float32),
            pltpu.VMEM((2, H), jnp.float32),
            pltpu.VMEM((2, H), jnp.float32),
            pltpu.VMEM((2, Cout), jnp.float32),
            pltpu.VMEM((H, H + Cout), jnp.bfloat16),
            pltpu.VMEM((1, H), jnp.float32),
            pltpu.VMEM((H, 2 * Cout), jnp.bfloat16),
            pltpu.VMEM((1, Cout), jnp.float32),
            pltpu.VMEM((Cout, Cout), jnp.bfloat16),
            pltpu.VMEM((1, Cout), jnp.float32),
        ],
    )(
        xf, W1.astype(jnp.bfloat16), row(b1), row(g1), row(beta1),
        W2, row(b2), row(g2), row(beta2),
        W3, row(b3), row(g3), row(beta3),
        Wl, row(bl),
    )

    return out.reshape(B, N, Cout)


# speedup vs baseline: 2.2882x; 1.0644x over previous
import functools

import jax
import jax.numpy as jnp
from jax.experimental import pallas as pl
from jax.experimental.pallas import tpu as pltpu

_EPS = 1e-5


def _affine_cols(s_scr, g_ref, beta_ref, m):
    mean = s_scr[0:1, :] / m
    var = s_scr[1:2, :] / m - mean * mean
    inv = jax.lax.rsqrt(var + _EPS)
    a = g_ref[...] * inv
    c = beta_ref[...] - mean * a
    return a, c


def _accum_stats(y, s_scr, i):
    part = jnp.concatenate(
        [jnp.sum(y, axis=0, keepdims=True), jnp.sum(y * y, axis=0, keepdims=True)],
        axis=0,
    )

    @pl.when(i == 0)
    def _():
        s_scr[...] = part

    @pl.when(i != 0)
    def _():
        s_scr[...] += part


def _fused_body(
    x_ref, W1_ref, b1_ref, g1_ref, beta1_ref, W2_ref, b2_ref, g2_ref, beta2_ref,
    W3_ref, b3_ref, g3_ref, beta3_ref, Wl_ref, bl_ref,
    out_ref,
    y1_scr, y2_scr, y3_scr, acc_scr, s1_scr, s2_scr, s3_scr,
    Wc1_scr, bp2_scr, Wc2_scr, bp3_scr, Wp3l_scr, bpl_scr,
    *, m, h, co, r,
):
    s = pl.program_id(0)
    i = pl.program_id(1)
    rows = pl.ds(i * r, r)

    @pl.when(s == 0)
    def _():
        xb = x_ref[...].astype(jnp.bfloat16)
        y = jax.nn.relu(
            jnp.dot(xb, W1_ref[...], preferred_element_type=jnp.float32)
            + b1_ref[...]
        )
        y1_scr[rows, :] = y.astype(jnp.bfloat16)
        _accum_stats(y, s1_scr, i)

    @pl.when((s == 1) & (i == 0))
    def _():
        a, c = _affine_cols(s1_scr, g1_ref, beta1_ref, m)
        av = a.reshape(-1, 1)
        Wc1_scr[:, 0:h] = (W2_ref[...] * av).astype(jnp.bfloat16)
        Wc1_scr[:, h:] = (Wl_ref[0:h, :] * av).astype(jnp.bfloat16)
        bp2_scr[...] = (
            jnp.dot(c, W2_ref[...], preferred_element_type=jnp.float32)
            + b2_ref[...]
        )

    @pl.when(s == 1)
    def _():
        z = jnp.dot(
            y1_scr[rows, :], Wc1_scr[...], preferred_element_type=jnp.float32
        )
        y = jax.nn.relu(z[:, 0:h] + bp2_scr[...])
        y2_scr[rows, :] = y.astype(jnp.bfloat16)
        acc_scr[rows, :] = z[:, h:].astype(jnp.bfloat16)
        _accum_stats(y, s2_scr, i)

    @pl.when((s == 2) & (i == 0))
    def _():
        a, c = _affine_cols(s2_scr, g2_ref, beta2_ref, m)
        av = a.reshape(-1, 1)
        Wc2_scr[:, 0:co] = (W3_ref[...] * av).astype(jnp.bfloat16)
        Wc2_scr[:, co:] = (Wl_ref[h : 2 * h, :] * av).astype(jnp.bfloat16)
        bp3_scr[...] = (
            jnp.dot(c, W3_ref[...], preferred_element_type=jnp.float32)
            + b3_ref[...]
        )

    @pl.when(s == 2)
    def _():
        z = jnp.dot(
            y2_scr[rows, :], Wc2_scr[...], preferred_element_type=jnp.float32
        )
        y = jax.nn.relu(z[:, 0:co] + bp3_scr[...])
        y3_scr[rows, :] = y.astype(jnp.bfloat16)
        acc_scr[rows, :] = (
            acc_scr[rows, :].astype(jnp.float32) + z[:, co:]
        ).astype(jnp.bfloat16)
        _accum_stats(y, s3_scr, i)

    @pl.when((s == 3) & (i == 0))
    def _():
        a1, c1 = _affine_cols(s1_scr, g1_ref, beta1_ref, m)
        a2, c2 = _affine_cols(s2_scr, g2_ref, beta2_ref, m)
        a3, c3 = _affine_cols(s3_scr, g3_ref, beta3_ref, m)
        Wp3l_scr[...] = (Wl_ref[2 * h :, :] * a3.reshape(-1, 1)).astype(jnp.bfloat16)
        bpl_scr[...] = (
            jnp.dot(c1, Wl_ref[0:h, :], preferred_element_type=jnp.float32)
            + jnp.dot(c2, Wl_ref[h : 2 * h, :], preferred_element_type=jnp.float32)
            + jnp.dot(c3, Wl_ref[2 * h :, :], preferred_element_type=jnp.float32)
            + bl_ref[...]
        )

    @pl.when(s == 3)
    def _():
        z = jnp.dot(
            y3_scr[rows, :], Wp3l_scr[...], preferred_element_type=jnp.float32
        )
        out_ref[...] = jax.nn.relu(
            acc_scr[rows, :].astype(jnp.float32) + z + bpl_scr[...]
        )


def kernel(x, adj, W1, b1, g1, beta1, W2, b2, g2, beta2, W3, b3, g3, beta3, Wl, bl):
    del adj
    B, N, Cin = x.shape
    H = W1.shape[1]
    Cout = W3.shape[1]
    M = B * N
    R = 10240
    nb = M // R
    mf = float(M)

    xf = x.reshape(M, Cin)
    row = lambda v: v.reshape(1, -1)
    full = lambda shape: pl.BlockSpec(shape, lambda s, i: (0, 0))

    out = pl.pallas_call(
        functools.partial(_fused_body, m=mf, h=H, co=Cout, r=R),
        grid=(4, nb),
        in_specs=[
            pl.BlockSpec((R, Cin), lambda s, i: (jnp.where(s == 0, i, 0), 0)),
            full((Cin, H)), full((1, H)), full((1, H)), full((1, H)),
            full((H, H)), full((1, H)), full((1, H)), full((1, H)),
            full((H, Cout)), full((1, Cout)), full((1, Cout)), full((1, Cout)),
            full((2 * H + Cout, Cout)), full((1, Cout)),
        ],
        out_specs=pl.BlockSpec((R, Cout), lambda s, i: (jnp.where(s == 3, i, 0), 0)),
        out_shape=jax.ShapeDtypeStruct((M, Cout), jnp.float32),
        scratch_shapes=[
            pltpu.VMEM((M, H), jnp.bfloat16),
            pltpu.VMEM((M, H), jnp.bfloat16),
            pltpu.VMEM((M, Cout), jnp.bfloat16),
            pltpu.VMEM((M, Cout), jnp.bfloat16),
            pltpu.VMEM((2, H), jnp.float32),
            pltpu.VMEM((2, H), jnp.float32),
            pltpu.VMEM((2, Cout), jnp.float32),
            pltpu.VMEM((H, H + Cout), jnp.bfloat16),
            pltpu.VMEM((1, H), jnp.float32),
            pltpu.VMEM((H, 2 * Cout), jnp.bfloat16),
            pltpu.VMEM((1, Cout), jnp.float32),
            pltpu.VMEM((Cout, Cout), jnp.bfloat16),
            pltpu.VMEM((1, Cout), jnp.float32),
        ],
    )(
        xf, W1.astype(jnp.bfloat16), row(b1), row(g1), row(beta1),
        W2, row(b2), row(g2), row(beta2),
        W3, row(b3), row(g3), row(beta3),
        Wl, row(bl),
    )

    return out.reshape(B, N, Cout)
